# Initial kernel scaffold; baseline (speedup 1.0000x reference)
import jax
import jax.numpy as jnp
from jax.experimental import pallas as pl


def kernel(x, edge_index, W_l, b_l, W_r, gamma, beta):
    raise NotImplementedError("write your pallas kernel here")


# speedup vs baseline: 3.8167x; 3.8167x over previous
import functools

import jax
import jax.numpy as jnp
from jax import lax
from jax.experimental import pallas as pl
from jax.experimental.pallas import tpu as pltpu
from jax.experimental.pallas import tpu_sc as plsc

_NC = 2
_NS = 16
_L = 16
_CHUNK = 128
_DG = 128


def _sc_segment_sum(x_stacked, src2, dst, zeros_h, n_pad, e_pad, h):
  t_per_tile = e_pad // _NS
  n_chunks = t_per_tile // _CHUNK
  r_per_tile = n_pad // _NS
  dg_per_tile = n_pad // _NS

  mesh = plsc.VectorSubcoreMesh(core_axis_name="c", subcore_axis_name="s")

  @functools.partial(
      pl.kernel,
      out_type=[
          jax.ShapeDtypeStruct((_NC * n_pad, h), jnp.float32),
          jax.ShapeDtypeStruct((_NC * n_pad,), jnp.float32),
      ],
      mesh=mesh,
      scratch_types=[
          pltpu.VMEM_SHARED((n_pad, h), jnp.float32),
          pltpu.VMEM_SHARED((n_pad,), jnp.float32),
          pltpu.VMEM((_CHUNK,), jnp.float32),
          pltpu.VMEM((n_pad // _NS,), jnp.float32),
          pltpu.VMEM((_CHUNK,), jnp.int32),
          pltpu.VMEM((_CHUNK,), jnp.int32),
          pltpu.VMEM((_CHUNK, h), jnp.float32),
          pltpu.SemaphoreType.DMA,
      ],
  )
  def sc_kernel(x_hbm, src2_hbm, dst_hbm, zh_hbm,
                agg_out, deg_out, agg_sh, deg_sh, ones_v, dstr_v,
                sidx_v, didx_v, rows_v, sem):
    cc = lax.axis_index("c")
    ss = lax.axis_index("s")

    row0 = ss * r_per_tile
    drow0 = ss * dg_per_tile
    pltpu.sync_copy(zh_hbm.at[pl.ds(row0, r_per_tile)],
                    agg_sh.at[pl.ds(row0, r_per_tile)])
    for g in range(_CHUNK // _L):
      ones_v[pl.ds(g * _L, _L)] = jnp.ones((_L,), jnp.float32)
    z16 = jnp.zeros((_L,), jnp.float32)

    def zbody(r, carry):
      dstr_v[pl.ds(r * _L, _L)] = z16
      return carry

    lax.fori_loop(0, (n_pad // _NS) // _L, zbody, 0)
    pltpu.sync_copy(dstr_v, deg_sh.at[pl.ds(drow0, n_pad // _NS)])
    plsc.subcore_barrier()

    base = cc * e_pad + ss * t_per_tile
    dbase = ss * t_per_tile
    ones16 = jnp.ones((_L,), jnp.float32)

    def body(i, carry):
      off = i * _CHUNK
      pltpu.sync_copy(src2_hbm.at[pl.ds(base + off, _CHUNK)], sidx_v)
      pltpu.sync_copy(dst_hbm.at[pl.ds(dbase + off, _CHUNK)], didx_v)
      pltpu.async_copy(x_hbm.at[sidx_v], rows_v, sem).wait()
      pltpu.sync_copy(rows_v, agg_sh.at[didx_v], add=True)
      pltpu.sync_copy(ones_v, deg_sh.at[didx_v], add=True)
      return carry

    lax.fori_loop(0, n_chunks, body, 0)

    plsc.subcore_barrier()

    pltpu.sync_copy(agg_sh.at[pl.ds(row0, r_per_tile)],
                    agg_out.at[pl.ds(cc * n_pad + row0, r_per_tile)])
    pltpu.sync_copy(deg_sh.at[pl.ds(drow0, n_pad // _NS)], dstr_v)
    pltpu.sync_copy(dstr_v, deg_out.at[pl.ds(cc * n_pad + drow0, n_pad // _NS)])

  return sc_kernel(x_stacked, src2, dst, zeros_h)


def _tc_epilogue(agg0, agg1, x, deg, wl_t, wr_t, b_l, gamma, beta, block_n):
  n, d = x.shape
  grid = (n // block_n,)

  def body(a0, a1, xr, dg, wl, wr, bl, gm, bt, out):
    degc = jnp.maximum(dg[...], 1.0)
    agg = jnp.concatenate([a0[...], a1[...]], axis=1) / degc
    h = (jnp.dot(agg, wl[...], preferred_element_type=jnp.float32)
         + bl[...]
         + jnp.dot(xr[...], wr[...], preferred_element_type=jnp.float32))
    nrm = jnp.sqrt(jnp.sum(h * h, axis=1, keepdims=True))
    h = h / jnp.maximum(nrm, 1e-12) + xr[...]
    mu = jnp.mean(h, axis=1, keepdims=True)
    var = jnp.mean((h - mu) * (h - mu), axis=1, keepdims=True)
    h = (h - mu) * lax.rsqrt(var + 1e-5) * gm[...] + bt[...]
    out[...] = 0.5 * h * (1.0 + lax.erf(h * 0.7071067811865476))

  hh = d // 2
  return pl.pallas_call(
      body,
      grid=grid,
      in_specs=[
          pl.BlockSpec((block_n, hh), lambda i: (i, 0)),
          pl.BlockSpec((block_n, hh), lambda i: (i, 0)),
          pl.BlockSpec((block_n, d), lambda i: (i, 0)),
          pl.BlockSpec((block_n, 1), lambda i: (i, 0)),
          pl.BlockSpec((d, d), lambda i: (0, 0)),
          pl.BlockSpec((d, d), lambda i: (0, 0)),
          pl.BlockSpec((1, d), lambda i: (0, 0)),
          pl.BlockSpec((1, d), lambda i: (0, 0)),
          pl.BlockSpec((1, d), lambda i: (0, 0)),
      ],
      out_specs=pl.BlockSpec((block_n, d), lambda i: (i, 0)),
      out_shape=jax.ShapeDtypeStruct((n, d), jnp.float32),
  )(agg0, agg1, x, deg, wl_t, wr_t, b_l, gamma, beta)


def kernel(x, edge_index, W_l, b_l, W_r, gamma, beta):
  n, d = x.shape
  e = edge_index.shape[1]
  h = d // 2

  n_pad = ((n + 1 + _NS * 8 - 1) // (_NS * 8)) * (_NS * 8)
  e_pad = ((e + _NS * _CHUNK - 1) // (_NS * _CHUNK)) * (_NS * _CHUNK)

  src = edge_index[0]
  dst = edge_index[1]
  pad_e = e_pad - e
  src_p = jnp.concatenate([src, jnp.zeros((pad_e,), jnp.int32)])
  dst_p = jnp.concatenate([dst, jnp.full((pad_e,), n, jnp.int32)])
  src2 = jnp.concatenate([src_p, src_p + n])

  x_stacked = jnp.concatenate([x[:, :h], x[:, h:]], axis=0)

  zeros_h = jnp.zeros((n_pad, h), jnp.float32)

  agg2, deg2 = _sc_segment_sum(x_stacked, src2, dst_p, zeros_h,
                               n_pad, e_pad, h)

  agg0 = agg2[:n]
  agg1 = agg2[n_pad:n_pad + n]
  deg = deg2[:n].reshape(n, 1)

  return _tc_epilogue(agg0, agg1, x, deg, W_l.T, W_r.T,
                      b_l.reshape(1, d), gamma.reshape(1, d),
                      beta.reshape(1, d), block_n=400)


# speedup vs baseline: 6.6047x; 1.7305x over previous
import functools

import jax
import jax.numpy as jnp
from jax import lax
from jax.experimental import pallas as pl
from jax.experimental.pallas import tpu as pltpu
from jax.experimental.pallas import tpu_sc as plsc

_NC = 2
_NS = 16
_L = 16
_CHUNK = 128


def _sc_segment_sum(x_stacked, src3, dst3, zeros_h, n_pad, e_pad, h):
  n_chunks = e_pad // (_NS * _CHUNK)
  r_per_tile = n_pad // _NS
  d_per_tile = n_pad // _NS

  mesh = plsc.VectorSubcoreMesh(core_axis_name="c", subcore_axis_name="s")

  @functools.partial(
      pl.kernel,
      out_type=[
          jax.ShapeDtypeStruct((_NC * n_pad, h), jnp.float32),
          jax.ShapeDtypeStruct((_NC * n_pad,), jnp.float32),
      ],
      mesh=mesh,
      scratch_types=[
          pltpu.VMEM_SHARED((n_pad, h), jnp.float32),
          pltpu.VMEM_SHARED((n_pad,), jnp.float32),
          pltpu.VMEM((_CHUNK,), jnp.int32),
          pltpu.VMEM((_CHUNK,), jnp.int32),
          pltpu.VMEM((_CHUNK,), jnp.int32),
          pltpu.VMEM((_CHUNK,), jnp.int32),
          pltpu.VMEM((_CHUNK, h), jnp.float32),
          pltpu.VMEM((_CHUNK, h), jnp.float32),
          pltpu.VMEM((_CHUNK,), jnp.float32),
          pltpu.VMEM((n_pad // _NS,), jnp.float32),
          pltpu.SemaphoreType.DMA,
          pltpu.SemaphoreType.DMA,
      ],
  )
  def sc_kernel(x_hbm, src3_hbm, dst3_hbm, zh_hbm, agg_out, deg_out,
                agg_sh, deg_sh, sbuf0_v, sbuf1_v, dbuf0_v, dbuf1_v,
                rows0_v, rows1_v, ones_v, dstr_v, gsem0, gsem1):
    cc = lax.axis_index("c")
    ss = lax.axis_index("s")
    srow0 = (cc * _NS + ss) * n_chunks
    drow_base = ss * n_chunks

    row0 = ss * r_per_tile
    drow0 = ss * d_per_tile
    pltpu.sync_copy(zh_hbm.at[pl.ds(row0, r_per_tile)],
                    agg_sh.at[pl.ds(row0, r_per_tile)])
    for g in range(_CHUNK // _L):
      ones_v[pl.ds(g * _L, _L)] = jnp.ones((_L,), jnp.float32)
    z16 = jnp.zeros((_L,), jnp.float32)

    def zbody(r, carry):
      dstr_v[pl.ds(r * _L, _L)] = z16
      return carry

    lax.fori_loop(0, d_per_tile // _L, zbody, 0)
    pltpu.sync_copy(dstr_v, deg_sh.at[pl.ds(drow0, d_per_tile)])
    plsc.subcore_barrier()

    def gather(i, sbuf, dbuf, rows, sem):
      pltpu.sync_copy(src3_hbm.at[srow0 + i], sbuf)
      pltpu.sync_copy(dst3_hbm.at[drow_base + i], dbuf)
      pltpu.async_copy(x_hbm.at[sbuf], rows, sem)

    def step(sbuf, dbuf, rows, sem, count_core):
      pltpu.make_async_copy(x_hbm.at[sbuf], rows, sem).wait()
      pltpu.sync_copy(rows, agg_sh.at[dbuf], add=True)

      @pl.when(cc == count_core)
      def _():
        pltpu.sync_copy(ones_v, deg_sh.at[dbuf], add=True)

    gather(0, sbuf0_v, dbuf0_v, rows0_v, gsem0)
    gather(1, sbuf1_v, dbuf1_v, rows1_v, gsem1)

    def body(gp, carry):
      i0 = 2 * gp
      step(sbuf0_v, dbuf0_v, rows0_v, gsem0, 0)
      gather(i0 + 2, sbuf0_v, dbuf0_v, rows0_v, gsem0)
      step(sbuf1_v, dbuf1_v, rows1_v, gsem1, 1)
      gather(i0 + 3, sbuf1_v, dbuf1_v, rows1_v, gsem1)
      return carry

    lax.fori_loop(0, n_chunks // 2 - 1, body, 0)
    step(sbuf0_v, dbuf0_v, rows0_v, gsem0, 0)
    step(sbuf1_v, dbuf1_v, rows1_v, gsem1, 1)
    plsc.subcore_barrier()

    pltpu.sync_copy(agg_sh.at[pl.ds(row0, r_per_tile)],
                    agg_out.at[pl.ds(cc * n_pad + row0, r_per_tile)])
    pltpu.sync_copy(deg_sh.at[pl.ds(drow0, d_per_tile)], dstr_v)
    pltpu.sync_copy(dstr_v,
                    deg_out.at[pl.ds(cc * n_pad + drow0, d_per_tile)])

  return sc_kernel(x_stacked, src3, dst3, zeros_h)


def _tc_epilogue(agg0, agg1, x, deg0, deg1, wl_t, wr_t, b_l, gamma, beta,
                 block_n):
  n, d = x.shape
  grid = (n // block_n,)

  def body(a0, a1, xr, dg0, dg1, wl, wr, bl, gm, bt, out):
    degc = jnp.maximum(dg0[...] + dg1[...], 1.0)
    agg = jnp.concatenate([a0[...], a1[...]], axis=1) / degc
    h = (jnp.dot(agg, wl[...], preferred_element_type=jnp.float32)
         + bl[...]
         + jnp.dot(xr[...], wr[...], preferred_element_type=jnp.float32))
    nrm = jnp.sqrt(jnp.sum(h * h, axis=1, keepdims=True))
    h = h / jnp.maximum(nrm, 1e-12) + xr[...]
    mu = jnp.mean(h, axis=1, keepdims=True)
    var = jnp.mean((h - mu) * (h - mu), axis=1, keepdims=True)
    h = (h - mu) * lax.rsqrt(var + 1e-5) * gm[...] + bt[...]
    out[...] = 0.5 * h * (1.0 + lax.erf(h * 0.7071067811865476))

  hh = d // 2
  return pl.pallas_call(
      body,
      grid=grid,
      in_specs=[
          pl.BlockSpec((block_n, hh), lambda i: (i, 0)),
          pl.BlockSpec((block_n, hh), lambda i: (i, 0)),
          pl.BlockSpec((block_n, d), lambda i: (i, 0)),
          pl.BlockSpec((block_n, 1), lambda i: (i, 0)),
          pl.BlockSpec((block_n, 1), lambda i: (i, 0)),
          pl.BlockSpec((d, d), lambda i: (0, 0)),
          pl.BlockSpec((d, d), lambda i: (0, 0)),
          pl.BlockSpec((1, d), lambda i: (0, 0)),
          pl.BlockSpec((1, d), lambda i: (0, 0)),
          pl.BlockSpec((1, d), lambda i: (0, 0)),
      ],
      out_specs=pl.BlockSpec((block_n, d), lambda i: (i, 0)),
      out_shape=jax.ShapeDtypeStruct((n, d), jnp.float32),
  )(agg0, agg1, x, deg0, deg1, wl_t, wr_t, b_l, gamma, beta)


def kernel(x, edge_index, W_l, b_l, W_r, gamma, beta):
  n, d = x.shape
  e = edge_index.shape[1]
  h = d // 2

  n_pad = ((n + 1 + _NS * 8 - 1) // (_NS * 8)) * (_NS * 8)
  step_e = _NS * _CHUNK * 2
  e_pad = ((e + step_e - 1) // step_e) * step_e
  n_chunks = e_pad // (_NS * _CHUNK)

  src = edge_index[0]
  dst = edge_index[1]
  pad_e = e_pad - e
  pad_idx = jnp.arange(pad_e, dtype=jnp.int32)
  src_p = jnp.concatenate([src, pad_idx % n])
  dst_p = jnp.concatenate([dst, n + pad_idx % (n_pad - n)])
  src2 = jnp.concatenate([src_p, src_p + n])
  src3 = src2.reshape(_NC * _NS * n_chunks, _CHUNK)
  dst3 = dst_p.reshape(_NS * n_chunks, _CHUNK)

  x_stacked = jnp.concatenate([x[:, :h], x[:, h:]], axis=0)

  zeros_h = jnp.zeros((n_pad, h), jnp.float32)

  agg2, deg2 = _sc_segment_sum(x_stacked, src3, dst3, zeros_h,
                               n_pad, e_pad, h)

  agg0 = agg2[:n]
  agg1 = agg2[n_pad:n_pad + n]
  deg0 = deg2[:n].reshape(n, 1)
  deg1 = deg2[n_pad:n_pad + n].reshape(n, 1)

  return _tc_epilogue(agg0, agg1, x, deg0, deg1, W_l.T, W_r.T,
                      b_l.reshape(1, d), gamma.reshape(1, d),
                      beta.reshape(1, d), block_n=400)


# speedup vs baseline: 6.8345x; 1.0348x over previous
import functools

import jax
import jax.numpy as jnp
from jax import lax
from jax.experimental import pallas as pl
from jax.experimental.pallas import tpu as pltpu
from jax.experimental.pallas import tpu_sc as plsc

_NC = 2
_NS = 16
_L = 16
_CHUNK = 128


def _sc_segment_sum(x_rows, src3, dst3, zeros_h, n_pad, e_pad, h):
  n_chunks = e_pad // (_NS * _CHUNK)
  r_per_tile = n_pad // _NS
  d_per_tile = n_pad // _NS

  mesh = plsc.VectorSubcoreMesh(core_axis_name="c", subcore_axis_name="s")

  @functools.partial(
      pl.kernel,
      out_type=[
          jax.ShapeDtypeStruct((n_pad, _NC * h), jnp.float32),
          jax.ShapeDtypeStruct((_NC * n_pad,), jnp.float32),
      ],
      mesh=mesh,
      scratch_types=[
          pltpu.VMEM_SHARED((n_pad, h), jnp.float32),
          pltpu.VMEM_SHARED((n_pad,), jnp.float32),
          pltpu.VMEM((_CHUNK,), jnp.int32),
          pltpu.VMEM((_CHUNK,), jnp.int32),
          pltpu.VMEM((_CHUNK,), jnp.int32),
          pltpu.VMEM((_CHUNK,), jnp.int32),
          pltpu.VMEM((_CHUNK, h), jnp.float32),
          pltpu.VMEM((_CHUNK, h), jnp.float32),
          pltpu.VMEM((_CHUNK,), jnp.float32),
          pltpu.VMEM((n_pad // _NS,), jnp.float32),
          pltpu.SemaphoreType.DMA,
          pltpu.SemaphoreType.DMA,
      ],
  )
  def sc_kernel(x_hbm, src3_hbm, dst3_hbm, zh_hbm, agg_out, deg_out,
                agg_sh, deg_sh, sbuf0_v, sbuf1_v, dbuf0_v, dbuf1_v,
                rows0_v, rows1_v, ones_v, dstr_v, gsem0, gsem1):
    cc = lax.axis_index("c")
    ss = lax.axis_index("s")
    srow0 = (cc * _NS + ss) * n_chunks
    drow_base = ss * n_chunks

    row0 = ss * r_per_tile
    drow0 = ss * d_per_tile
    pltpu.sync_copy(zh_hbm.at[pl.ds(row0, r_per_tile)],
                    agg_sh.at[pl.ds(row0, r_per_tile)])
    for g in range(_CHUNK // _L):
      ones_v[pl.ds(g * _L, _L)] = jnp.ones((_L,), jnp.float32)
    z16 = jnp.zeros((_L,), jnp.float32)

    def zbody(r, carry):
      dstr_v[pl.ds(r * _L, _L)] = z16
      return carry

    lax.fori_loop(0, d_per_tile // _L, zbody, 0)
    pltpu.sync_copy(dstr_v, deg_sh.at[pl.ds(drow0, d_per_tile)])
    plsc.subcore_barrier()

    def gather(i, sbuf, dbuf, rows, sem):
      pltpu.sync_copy(src3_hbm.at[srow0 + i], sbuf)
      pltpu.sync_copy(dst3_hbm.at[drow_base + i], dbuf)
      pltpu.async_copy(x_hbm.at[sbuf], rows, sem)

    def step(sbuf, dbuf, rows, sem, count_core):
      pltpu.make_async_copy(x_hbm.at[sbuf], rows, sem).wait()
      pltpu.sync_copy(rows, agg_sh.at[dbuf], add=True)

      @pl.when(cc == count_core)
      def _():
        pltpu.sync_copy(ones_v, deg_sh.at[dbuf], add=True)

    gather(0, sbuf0_v, dbuf0_v, rows0_v, gsem0)
    gather(1, sbuf1_v, dbuf1_v, rows1_v, gsem1)

    def body(gp, carry):
      i0 = 2 * gp
      step(sbuf0_v, dbuf0_v, rows0_v, gsem0, 0)
      gather(i0 + 2, sbuf0_v, dbuf0_v, rows0_v, gsem0)
      step(sbuf1_v, dbuf1_v, rows1_v, gsem1, 1)
      gather(i0 + 3, sbuf1_v, dbuf1_v, rows1_v, gsem1)
      return carry

    lax.fori_loop(0, n_chunks // 2 - 1, body, 0)
    step(sbuf0_v, dbuf0_v, rows0_v, gsem0, 0)
    step(sbuf1_v, dbuf1_v, rows1_v, gsem1, 1)
    plsc.subcore_barrier()

    pltpu.sync_copy(agg_sh.at[pl.ds(row0, r_per_tile)],
                    agg_out.at[pl.ds(row0, r_per_tile), pl.ds(cc * h, h)])
    pltpu.sync_copy(deg_sh.at[pl.ds(drow0, d_per_tile)], dstr_v)
    pltpu.sync_copy(dstr_v,
                    deg_out.at[pl.ds(cc * n_pad + drow0, d_per_tile)])

  return sc_kernel(x_rows, src3, dst3, zeros_h)


def _tc_epilogue(agg, x, deg0, deg1, wl_t, wr_t, b_l, gamma, beta,
                 block_n):
  n, d = x.shape
  grid = (n // block_n,)

  def body(ag, xr, dg0, dg1, wl, wr, bl, gm, bt, out):
    degc = jnp.maximum(dg0[...] + dg1[...], 1.0)
    agg = ag[...] / degc
    h = (jnp.dot(agg, wl[...], preferred_element_type=jnp.float32)
         + bl[...]
         + jnp.dot(xr[...], wr[...], preferred_element_type=jnp.float32))
    nrm = jnp.sqrt(jnp.sum(h * h, axis=1, keepdims=True))
    h = h / jnp.maximum(nrm, 1e-12) + xr[...]
    mu = jnp.mean(h, axis=1, keepdims=True)
    var = jnp.mean((h - mu) * (h - mu), axis=1, keepdims=True)
    h = (h - mu) * lax.rsqrt(var + 1e-5) * gm[...] + bt[...]
    out[...] = 0.5 * h * (1.0 + lax.erf(h * 0.7071067811865476))

  return pl.pallas_call(
      body,
      grid=grid,
      in_specs=[
          pl.BlockSpec((block_n, d), lambda i: (i, 0)),
          pl.BlockSpec((block_n, d), lambda i: (i, 0)),
          pl.BlockSpec((block_n, 1), lambda i: (i, 0)),
          pl.BlockSpec((block_n, 1), lambda i: (i, 0)),
          pl.BlockSpec((d, d), lambda i: (0, 0)),
          pl.BlockSpec((d, d), lambda i: (0, 0)),
          pl.BlockSpec((1, d), lambda i: (0, 0)),
          pl.BlockSpec((1, d), lambda i: (0, 0)),
          pl.BlockSpec((1, d), lambda i: (0, 0)),
      ],
      out_specs=pl.BlockSpec((block_n, d), lambda i: (i, 0)),
      out_shape=jax.ShapeDtypeStruct((n, d), jnp.float32),
  )(agg, x, deg0, deg1, wl_t, wr_t, b_l, gamma, beta)


def kernel(x, edge_index, W_l, b_l, W_r, gamma, beta):
  n, d = x.shape
  e = edge_index.shape[1]
  h = d // 2

  n_pad = ((n + 1 + _NS * 8 - 1) // (_NS * 8)) * (_NS * 8)
  step_e = _NS * _CHUNK * 2
  e_pad = ((e + step_e - 1) // step_e) * step_e
  n_chunks = e_pad // (_NS * _CHUNK)

  src = edge_index[0]
  dst = edge_index[1]
  pad_e = e_pad - e
  pad_idx = jnp.arange(pad_e, dtype=jnp.int32)
  src_p = jnp.concatenate([src, pad_idx % n])
  dst_p = jnp.concatenate([dst, n + pad_idx % (n_pad - n)])
  src_p2 = src_p * 2
  src2 = jnp.concatenate([src_p2, src_p2 + 1])
  src3 = src2.reshape(_NC * _NS * n_chunks, _CHUNK)
  dst3 = dst_p.reshape(_NS * n_chunks, _CHUNK)

  x_rows = x.reshape(2 * n, h)

  zeros_h = jnp.zeros((n_pad, h), jnp.float32)

  agg, deg2 = _sc_segment_sum(x_rows, src3, dst3, zeros_h,
                              n_pad, e_pad, h)

  deg0 = deg2[:n].reshape(n, 1)
  deg1 = deg2[n_pad:n_pad + n].reshape(n, 1)

  return _tc_epilogue(agg, x, deg0, deg1, W_l.T, W_r.T,
                      b_l.reshape(1, d), gamma.reshape(1, d),
                      beta.reshape(1, d), block_n=400)
